# Initial kernel scaffold; baseline (speedup 1.0000x reference)
import jax
import jax.numpy as jnp
from jax.experimental import pallas as pl


def kernel(center, context, negatives, in_emb, out_emb):
    raise NotImplementedError("write your pallas kernel here")


# speedup vs baseline: 4.1062x; 4.1062x over previous
import functools

import jax
import jax.numpy as jnp
from jax import lax
from jax.experimental import pallas as pl
from jax.experimental.pallas import tpu as pltpu
from jax.experimental.pallas import tpu_sc as plsc

VOCAB = 1000000
DIM = 64
BATCH = 16384
NEG = 20
KP1 = NEG + 1

NC = 2
NS = 16
NW = NC * NS

EPW = BATCH // NW
C = 32
NCHUNK = EPW // C
CO = C * KP1

_mesh = plsc.VectorSubcoreMesh(core_axis_name="c", subcore_axis_name="s")


@functools.partial(
    pl.kernel,
    out_type=jax.ShapeDtypeStruct((BATCH * KP1,), jnp.float32),
    mesh=_mesh,
    compiler_params=pltpu.CompilerParams(needs_layout_passes=False),
    scratch_types=[
        pltpu.VMEM((C,), jnp.int32),
        pltpu.VMEM((CO,), jnp.int32),
        pltpu.VMEM((C,), jnp.int32),
        pltpu.VMEM((CO,), jnp.int32),
        pltpu.VMEM((C, 128), jnp.float32),
        pltpu.VMEM((CO, 128), jnp.float32),
        pltpu.VMEM((CO,), jnp.float32),
        pltpu.SemaphoreType.DMA,
    ],
)
def _sc_scores(center_hbm, out_idx_hbm, in_tbl_hbm, out_tbl_hbm,
               scores_hbm, cidx_v, oidx_v, cj_v, oj_v, crow_v, orow_v,
               sco_v, sem):
    wid = lax.axis_index("s") * NC + lax.axis_index("c")
    lane = lax.broadcasted_iota(jnp.int32, (16,), 0)

    def chunk_body(g, carry):
        base_e = wid * EPW + g * C
        pltpu.sync_copy(center_hbm.at[pl.ds(base_e, C)], cidx_v)
        pltpu.sync_copy(out_idx_hbm.at[pl.ds(base_e * KP1, CO)], oidx_v)

        for i in range(C // 16):
            cj_v[pl.ds(16 * i, 16)] = cidx_v[pl.ds(16 * i, 16)] >> 1
        for i in range(CO // 16):
            oj_v[pl.ds(16 * i, 16)] = oidx_v[pl.ds(16 * i, 16)] >> 1

        cps = [pltpu.async_copy(in_tbl_hbm.at[cj_v], crow_v, sem)]
        off = 0
        while off < CO:
            n = min(128, CO - off)
            cps.append(pltpu.async_copy(
                out_tbl_hbm.at[oj_v.at[pl.ds(off, n)]],
                orow_v.at[pl.ds(off, n)], sem))
            off += n
        for cp in cps:
            cp.wait()

        def group_body(gi, carry2):
            e0 = gi * 16
            ev = e0 + lane
            cidx16 = cidx_v[pl.ds(e0, 16)]
            ccol = (cidx16 & 1) << 6
            ev21 = ev * KP1
            acc = [jnp.zeros((16,), jnp.float32) for _ in range(KP1)]
            for db in range(8):
                d0 = db * 8
                c_regs = [
                    plsc.load_gather(crow_v, [ev, ccol + (d0 + t)])
                    for t in range(8)
                ]
                for k in range(KP1):
                    opos = ev21 + k
                    oidx16 = plsc.load_gather(oidx_v, [opos])
                    ocol = (oidx16 & 1) << 6
                    a = acc[k]
                    for t in range(8):
                        o = plsc.load_gather(orow_v, [opos, ocol + (d0 + t)])
                        a = a + c_regs[t] * o
                    acc[k] = a
            for k in range(KP1):
                sco_v[pl.ds(k * C + e0, 16)] = acc[k]
            return carry2

        lax.fori_loop(0, C // 16, group_body, 0)
        pltpu.sync_copy(sco_v, scores_hbm.at[pl.ds(base_e * KP1, CO)])
        return carry

    lax.fori_loop(0, NCHUNK, chunk_body, 0)


def _tc_loss_body(s_ref, o_ref):
    s = s_ref[...]
    rows, cols = s.shape
    r = lax.broadcasted_iota(jnp.int32, (rows, cols), 0)
    c = lax.broadcasted_iota(jnp.int32, (rows, cols), 1)
    p = r * cols + c
    is_pos = ((p // C) % KP1) == 0
    t = jnp.where(is_pos, s, -s)
    ls = jnp.minimum(t, 0.0) - jnp.log1p(jnp.exp(-jnp.abs(t)))
    o_ref[0, 0] = -jnp.sum(ls) / BATCH


_tc_loss = pl.pallas_call(
    _tc_loss_body,
    out_shape=jax.ShapeDtypeStruct((1, 1), jnp.float32),
    out_specs=pl.BlockSpec(memory_space=pltpu.SMEM),
)


def kernel(center, context, negatives, in_emb, out_emb):
    in_tbl = in_emb.reshape(VOCAB // 2, 2 * DIM)
    out_tbl = out_emb.reshape(VOCAB // 2, 2 * DIM)
    out_idx = jnp.concatenate([context[:, None], negatives], axis=1)
    out_idx = out_idx.reshape(BATCH * KP1)
    scores = _sc_scores(center, out_idx, in_tbl, out_tbl)
    loss = _tc_loss(scores.reshape(336, 1024))
    return loss.reshape(())


# speedup vs baseline: 4.8341x; 1.1773x over previous
import functools

import jax
import jax.numpy as jnp
from jax import lax
from jax.experimental import pallas as pl
from jax.experimental.pallas import tpu as pltpu
from jax.experimental.pallas import tpu_sc as plsc

VOCAB = 1000000
DIM = 64
BATCH = 16384
NEG = 20
KP1 = NEG + 1

NC = 2
NS = 16
NW = NC * NS

EPW = BATCH // NW
C = 32
NCHUNK = EPW // C
CO = C * KP1

_mesh = plsc.VectorSubcoreMesh(core_axis_name="c", subcore_axis_name="s")


@functools.partial(
    pl.kernel,
    out_type=jax.ShapeDtypeStruct((BATCH * KP1,), jnp.float32),
    mesh=_mesh,
    compiler_params=pltpu.CompilerParams(needs_layout_passes=False),
    scratch_types=[
        pltpu.VMEM((C,), jnp.int32),
        pltpu.VMEM((CO,), jnp.int32),
        pltpu.VMEM((C, 128), jnp.float32),
        pltpu.VMEM((CO, 128), jnp.float32),
        pltpu.VMEM((CO,), jnp.float32),
        pltpu.SemaphoreType.DMA,
    ],
)
def _sc_scores(center_hbm, out_idx_hbm, tbl_hbm,
               scores_hbm, cidx_v, oidx_v, crow_v, orow_v, sco_v, sem):
    wid = lax.axis_index("s") * NC + lax.axis_index("c")
    lane = lax.broadcasted_iota(jnp.int32, (16,), 0)

    def chunk_body(g, carry):
        base_e = wid * EPW + g * C
        pltpu.sync_copy(center_hbm.at[pl.ds(base_e, C)], cidx_v)
        pltpu.sync_copy(out_idx_hbm.at[pl.ds(base_e * KP1, CO)], oidx_v)

        cps = [pltpu.async_copy(tbl_hbm.at[cidx_v], crow_v, sem)]
        off = 0
        while off < CO:
            n = min(128, CO - off)
            cps.append(pltpu.async_copy(
                tbl_hbm.at[oidx_v.at[pl.ds(off, n)]],
                orow_v.at[pl.ds(off, n)], sem))
            off += n
        for cp in cps:
            cp.wait()

        def group_body(gi, carry2):
            e0 = gi * 16
            ev = e0 + lane
            ev21 = ev * KP1
            acc = [jnp.zeros((16,), jnp.float32) for _ in range(KP1)]
            for db in range(8):
                d0 = db * 8
                c_regs = [
                    plsc.load_gather(crow_v, [ev, jnp.full((16,), d0 + t,
                                                           jnp.int32)])
                    for t in range(8)
                ]
                for k in range(KP1):
                    opos = ev21 + k
                    a = acc[k]
                    for t in range(8):
                        o = plsc.load_gather(
                            orow_v, [opos, jnp.full((16,), 64 + d0 + t,
                                                    jnp.int32)])
                        a = a + c_regs[t] * o
                    acc[k] = a
            for k in range(KP1):
                sco_v[pl.ds(k * C + e0, 16)] = acc[k]
            return carry2

        lax.fori_loop(0, C // 16, group_body, 0)
        pltpu.sync_copy(sco_v, scores_hbm.at[pl.ds(base_e * KP1, CO)])
        return carry

    lax.fori_loop(0, NCHUNK, chunk_body, 0)


def _tc_loss_body(s_ref, o_ref):
    s = s_ref[...]
    rows, cols = s.shape
    r = lax.broadcasted_iota(jnp.int32, (rows, cols), 0)
    c = lax.broadcasted_iota(jnp.int32, (rows, cols), 1)
    p = r * cols + c
    is_pos = ((p // C) % KP1) == 0
    t = jnp.where(is_pos, s, -s)
    ls = jnp.minimum(t, 0.0) - jnp.log1p(jnp.exp(-jnp.abs(t)))
    o_ref[0, 0] = -jnp.sum(ls) / BATCH


_tc_loss = pl.pallas_call(
    _tc_loss_body,
    out_shape=jax.ShapeDtypeStruct((1, 1), jnp.float32),
    out_specs=pl.BlockSpec(memory_space=pltpu.SMEM),
)


def kernel(center, context, negatives, in_emb, out_emb):
    in_t, out_t = jax.lax.optimization_barrier((in_emb.T, out_emb.T))
    tbl = jnp.concatenate([in_t.T, out_t.T], axis=1)
    out_idx = jnp.concatenate([context[:, None], negatives], axis=1)
    out_idx = out_idx.reshape(BATCH * KP1)
    scores = _sc_scores(center, out_idx, tbl)
    loss = _tc_loss(scores.reshape(336, 1024))
    return loss.reshape(())


# speedup vs baseline: 4.9395x; 1.0218x over previous
import functools

import jax
import jax.numpy as jnp
from jax import lax
from jax.experimental import pallas as pl
from jax.experimental.pallas import tpu as pltpu
from jax.experimental.pallas import tpu_sc as plsc

VOCAB = 1000000
DIM = 64
BATCH = 16384
NEG = 20
KP1 = NEG + 1

NC = 2
NS = 16
NW = NC * NS

EPW = BATCH // NW
C = 16
NCHUNK = EPW // C
CO = C * KP1

O_SPLIT = [(0, 128), (128, 128), (256, CO - 256)]

_mesh = plsc.VectorSubcoreMesh(core_axis_name="c", subcore_axis_name="s")


@functools.partial(
    pl.kernel,
    out_type=jax.ShapeDtypeStruct((BATCH * KP1,), jnp.float32),
    mesh=_mesh,
    compiler_params=pltpu.CompilerParams(needs_layout_passes=False),
    scratch_types=[
        pltpu.VMEM((2 * C,), jnp.int32),
        pltpu.VMEM((2 * CO,), jnp.int32),
        pltpu.VMEM((2 * C, 128), jnp.float32),
        pltpu.VMEM((2 * CO, 128), jnp.float32),
        pltpu.VMEM((2 * CO,), jnp.float32),
        pltpu.SemaphoreType.DMA,
        pltpu.SemaphoreType.DMA,
    ],
)
def _sc_scores(center_hbm, out_idx_hbm, tbl_hbm, scores_hbm,
               cidx2, oidx2, crow2, orow2, sco2, sem_g, sem_s):
    wid = lax.axis_index("s") * NC + lax.axis_index("c")
    lane = lax.broadcasted_iota(jnp.int32, (16,), 0)
    ebase = wid * EPW

    def gather_copies(q):
        p = (q & 1) * C
        po = (q & 1) * CO
        cps = [pltpu.make_async_copy(
            tbl_hbm.at[cidx2.at[pl.ds(p, C)]],
            crow2.at[pl.ds(p, C)], sem_g)]
        for off, n in O_SPLIT:
            cps.append(pltpu.make_async_copy(
                tbl_hbm.at[oidx2.at[pl.ds(po + off, n)]],
                orow2.at[pl.ds(po + off, n)], sem_g))
        return cps

    def copy_idx(q):
        p = (q & 1) * C
        po = (q & 1) * CO
        b = ebase + q * C
        pltpu.sync_copy(center_hbm.at[pl.ds(b, C)], cidx2.at[pl.ds(p, C)])
        pltpu.sync_copy(out_idx_hbm.at[pl.ds(b * KP1, CO)],
                        oidx2.at[pl.ds(po, CO)])

    def score_write(q):
        po = (q & 1) * CO
        sb = (ebase + q * C) * KP1
        return pltpu.make_async_copy(
            sco2.at[pl.ds(po, CO)], scores_hbm.at[pl.ds(sb, CO)], sem_s)

    copy_idx(0)
    copy_idx(1)
    for cp in gather_copies(0):
        cp.start()

    def chunk_body(g, carry):
        p = (g & 1) * C
        po = (g & 1) * CO

        for cp in gather_copies(g):
            cp.wait()

        @pl.when(g + 1 < NCHUNK)
        def _():
            for cp in gather_copies(g + 1):
                cp.start()

        @pl.when(g + 2 < NCHUNK)
        def _():
            copy_idx(g + 2)

        @pl.when(g >= 2)
        def _():
            score_write(g - 2).wait()

        crow = p + lane
        orow = po + lane * KP1
        acc = [jnp.zeros((16,), jnp.float32) for _ in range(KP1)]
        for db in range(8):
            d0 = db * 8
            c_regs = [
                plsc.load_gather(
                    crow2, [crow, jnp.full((16,), d0 + t, jnp.int32)])
                for t in range(8)
            ]
            for k in range(KP1):
                opos = orow + k
                a = acc[k]
                for t in range(8):
                    o = plsc.load_gather(
                        orow2, [opos, jnp.full((16,), 64 + d0 + t,
                                               jnp.int32)])
                    a = a + c_regs[t] * o
                acc[k] = a
        for k in range(KP1):
            sco2[pl.ds(po + k * C, 16)] = acc[k]

        score_write(g).start()
        return carry

    lax.fori_loop(0, NCHUNK, chunk_body, 0)
    score_write(NCHUNK - 2).wait()
    score_write(NCHUNK - 1).wait()


def _tc_loss_body(s_ref, o_ref):
    s = s_ref[...]
    rows, cols = s.shape
    r = lax.broadcasted_iota(jnp.int32, (rows, cols), 0)
    c = lax.broadcasted_iota(jnp.int32, (rows, cols), 1)
    p = r * cols + c
    is_pos = ((p // C) % KP1) == 0
    t = jnp.where(is_pos, s, -s)
    ls = jnp.minimum(t, 0.0) - jnp.log1p(jnp.exp(-jnp.abs(t)))
    o_ref[0, 0] = -jnp.sum(ls) / BATCH


_tc_loss = pl.pallas_call(
    _tc_loss_body,
    out_shape=jax.ShapeDtypeStruct((1, 1), jnp.float32),
    out_specs=pl.BlockSpec(memory_space=pltpu.SMEM),
)


def kernel(center, context, negatives, in_emb, out_emb):
    tbl = jnp.concatenate([in_emb, out_emb], axis=1)
    out_idx = jnp.concatenate([context[:, None], negatives], axis=1)
    out_idx = out_idx.reshape(BATCH * KP1)
    scores = _sc_scores(center, out_idx, tbl)
    loss = _tc_loss(scores.reshape(336, 1024))
    return loss.reshape(())
